# baseline (speedup 1.0000x reference)
import jax
import jax.numpy as jnp
from jax import lax
from jax.experimental import pallas as pl
from jax.experimental.pallas import tpu as pltpu
from jax.experimental.pallas import tpu_sc as plsc

B, T, C, P, R = 16384, 32, 8, 3, 9
NC, NS, L = 2, 16, 16
NW = NC * NS
ROWS = B // NW
GROUPS = ROWS // L
NSF = ROWS * T * P
CHROWS = 64
NCHUNK = ROWS // CHROWS
CHW = CHROWS * T * C
PAR_RE = 32


def _softplus(x):
  t = jnp.exp(-jnp.abs(x))
  z = t / (t + 2.0)
  w = z * z
  poly = 1.0 + w * (1.0 / 3.0 + w * (1.0 / 5.0 + w * (1.0 / 7.0 + w * (
      1.0 / 9.0 + w * (1.0 / 11.0 + w * (1.0 / 13.0))))))
  return jnp.maximum(x, 0.0) + 2.0 * z * poly


def _body(ns_hbm, ch_hbm, rl_hbm, par_hbm, out_hbm,
          ns_v, ch_v, rl_v, par_v, out_v, slab0, slab1,
          sem_ns, sem0, sem1):
  cid = lax.axis_index("c")
  sid = lax.axis_index("s")
  wid = sid * NC + cid
  base = wid * ROWS

  lane = lax.broadcasted_iota(jnp.int32, (L,), 0)

  cp_ns = pltpu.async_copy(ns_hbm.at[pl.ds(base * T * P, NSF)], ns_v, sem_ns)
  pltpu.sync_copy(par_hbm, par_v)

  slabs = (slab0, slab1)
  sems = (sem0, sem1)

  def pull_column(src_hbm, dst_v):
    def start(k):
      off = (base + k * CHROWS) * T * C
      return pltpu.async_copy(src_hbm.at[pl.ds(off, CHW)], slabs[k % 2],
                              sems[k % 2])
    cps = {0: start(0)}
    for k in range(NCHUNK):
      if k + 1 < NCHUNK:
        cps[(k + 1) % 2] = start(k + 1)
      cps[k % 2].wait()
      slab = slabs[k % 2]
      kbase = k * CHROWS * T

      @plsc.parallel_loop(0, CHW // (L * C), unroll=4)
      def _extract(g):
        idx = g * (L * C) + lane * C + (C - 1)
        val = plsc.load_gather(slab, [idx])
        dst_v[pl.ds(kbase + g * L, L)] = val

  pull_column(ch_hbm, ch_v)
  pull_column(rl_hbm, rl_v)
  cp_ns.wait()

  wv = par_v[pl.ds(0, L)]
  vv = par_v[pl.ds(L, L)]
  w = [[wv[q * P + p] for p in range(P)] for q in range(P)]
  v = [[vv[q * P + p] for p in range(P)] for q in range(P)]

  @plsc.parallel_loop(0, GROUPS, unroll=2)
  def _group(g):
    rows = g * L + lane
    rowT = rows * T
    rowTP = rowT * P
    for i in range(T):
      c = plsc.load_gather(ch_v, [rowT + i])
      r = plsc.load_gather(rl_v, [rowT + i])
      pb = rowTP + i * P
      cb = rowTP + c * P
      rb = r * P + PAR_RE
      prnt = [plsc.load_gather(ns_v, [pb + p]) for p in range(P)]
      cs = [plsc.load_gather(ns_v, [cb + p]) for p in range(P)]
      re = [plsc.load_gather(par_v, [rb + p]) for p in range(P)]
      a = []
      for p in range(P):
        x = re[p]
        for q in range(P):
          x = x + w[q][p] * prnt[q]
          x = x + v[q][p] * cs[q]
        a.append(_softplus(x) + 1e-6)
      inv = 1.0 / (a[0] + a[1] + a[2])
      for p in range(P):
        plsc.store_scatter(ns_v, [pb + p], a[p] * inv)
    src = rowTP + (T - 1) * P
    dst = rows * P
    for p in range(P):
      val = plsc.load_gather(ns_v, [src + p])
      plsc.store_scatter(out_v, [dst + p], val)

  pltpu.sync_copy(out_v, out_hbm.at[pl.ds(base * P, ROWS * P)])


@jax.jit
def _run(ns_flat, ch_flat, rl_flat, params):
  mesh = plsc.VectorSubcoreMesh(core_axis_name="c", subcore_axis_name="s")
  f = pl.kernel(
      _body,
      out_type=jax.ShapeDtypeStruct((B * P,), jnp.float32),
      mesh=mesh,
      scratch_types=[
          pltpu.VMEM((NSF,), jnp.float32),
          pltpu.VMEM((ROWS * T,), jnp.int32),
          pltpu.VMEM((ROWS * T,), jnp.int32),
          pltpu.VMEM((64,), jnp.float32),
          pltpu.VMEM((ROWS * P,), jnp.float32),
          pltpu.VMEM((CHW,), jnp.int32),
          pltpu.VMEM((CHW,), jnp.int32),
          pltpu.SemaphoreType.DMA,
          pltpu.SemaphoreType.DMA,
          pltpu.SemaphoreType.DMA,
      ],
      compiler_params=pltpu.CompilerParams(
          needs_layout_passes=False, use_tc_tiling_on_sc=False),
  )
  return f(ns_flat, ch_flat, rl_flat, params)


def kernel(node_scores, children, rels, labels, W, V, rel_emb):
  del labels
  params = (jnp.zeros((64,), jnp.float32)
            .at[0:P * P].set(W.reshape(-1))
            .at[L:L + P * P].set(V.reshape(-1))
            .at[PAR_RE:PAR_RE + R * P].set(rel_emb.reshape(-1)))
  out = _run(node_scores.reshape(-1),
             children.reshape(-1).astype(jnp.int32),
             rels.reshape(-1).astype(jnp.int32),
             params)
  return out.reshape(B, P)


# speedup vs baseline: 14.2901x; 14.2901x over previous
import jax
import jax.numpy as jnp
from jax import lax
from jax.experimental import pallas as pl
from jax.experimental.pallas import tpu as pltpu
from jax.experimental.pallas import tpu_sc as plsc

B, T, C, P, R = 16384, 32, 8, 3, 9
NC, NS, L = 2, 16, 16
NW = NC * NS
ROWS = B // NW
GROUPS = ROWS // L
PAR_RE = 32


def _softplus(x):
  t = jnp.exp(-jnp.abs(x))
  z = t / (t + 2.0)
  w = z * z
  poly = 1.0 + w * (1.0 / 3.0 + w * (1.0 / 5.0 + w * (1.0 / 7.0)))
  return jnp.maximum(x, 0.0) + 2.0 * z * poly


def _body(ns_hbm, ch_hbm, rl_hbm, par_hbm, out_hbm,
          ns_v, ch_v, rl_v, par_v, sem_ns, sem_ch, sem_rl):
  cid = lax.axis_index("c")
  sid = lax.axis_index("s")
  wid = sid * NC + cid
  base = wid * ROWS

  lane = lax.broadcasted_iota(jnp.int32, (L,), 0)

  cp_ns = pltpu.async_copy(ns_hbm.at[:, :, pl.ds(base, ROWS)], ns_v, sem_ns)
  cp_ch = pltpu.async_copy(ch_hbm.at[:, C - 1, pl.ds(base, ROWS)], ch_v,
                           sem_ch)
  cp_rl = pltpu.async_copy(rl_hbm.at[:, C - 1, pl.ds(base, ROWS)], rl_v,
                           sem_rl)
  pltpu.sync_copy(par_hbm, par_v)
  cp_ns.wait()
  cp_ch.wait()
  cp_rl.wait()

  wv = par_v[pl.ds(0, L)]
  vv = par_v[pl.ds(L, L)]
  w = [[wv[q * P + p] for p in range(P)] for q in range(P)]
  v = [[vv[q * P + p] for p in range(P)] for q in range(P)]

  def i_step(i, carry):

    @plsc.parallel_loop(0, GROUPS, unroll=4)
    def _group(g):
      sl = pl.ds(g * L, L)
      rows = g * L + lane
      c = ch_v[i, sl]
      r = rl_v[i, sl]
      prnt = [ns_v[p, i, sl] for p in range(P)]
      cs = [plsc.load_gather(ns_v, [jnp.full((L,), p, jnp.int32), c, rows])
            for p in range(P)]
      re = [plsc.load_gather(par_v, [r * P + (PAR_RE + p)]) for p in range(P)]
      a = []
      for p in range(P):
        x = re[p]
        for q in range(P):
          x = x + w[q][p] * prnt[q]
          x = x + v[q][p] * cs[q]
        a.append(_softplus(x) + 1e-6)
      inv = 1.0 / (a[0] + a[1] + a[2])
      for p in range(P):
        ns_v[p, i, sl] = a[p] * inv

    return carry

  lax.fori_loop(0, T, i_step, 0)

  pltpu.sync_copy(ns_v.at[:, T - 1, :], out_hbm.at[:, pl.ds(base, ROWS)])


@jax.jit
def _run(ns_t, ch_t, rl_t, params):
  mesh = plsc.VectorSubcoreMesh(core_axis_name="c", subcore_axis_name="s")
  f = pl.kernel(
      _body,
      out_type=jax.ShapeDtypeStruct((P, B), jnp.float32),
      mesh=mesh,
      scratch_types=[
          pltpu.VMEM((P, T, ROWS), jnp.float32),
          pltpu.VMEM((T, ROWS), jnp.int32),
          pltpu.VMEM((T, ROWS), jnp.int32),
          pltpu.VMEM((64,), jnp.float32),
          pltpu.SemaphoreType.DMA,
          pltpu.SemaphoreType.DMA,
          pltpu.SemaphoreType.DMA,
      ],
      compiler_params=pltpu.CompilerParams(
          needs_layout_passes=False, use_tc_tiling_on_sc=False),
  )
  return f(ns_t, ch_t, rl_t, params)


def kernel(node_scores, children, rels, labels, W, V, rel_emb):
  del labels
  ns_t = jnp.transpose(node_scores, (2, 1, 0))
  ch_t = jnp.transpose(children, (1, 2, 0)).astype(jnp.int32)
  rl_t = jnp.transpose(rels, (1, 2, 0)).astype(jnp.int32)
  params = (jnp.zeros((64,), jnp.float32)
            .at[0:P * P].set(W.reshape(-1))
            .at[L:L + P * P].set(V.reshape(-1))
            .at[PAR_RE:PAR_RE + R * P].set(rel_emb.reshape(-1)))
  out = _run(ns_t, ch_t, rl_t, params)
  return jnp.transpose(out, (1, 0))


# speedup vs baseline: 25.6851x; 1.7974x over previous
import jax
import jax.numpy as jnp
from jax import lax
from jax.experimental import pallas as pl
from jax.experimental.pallas import tpu as pltpu
from jax.experimental.pallas import tpu_sc as plsc

B, T, C, P, R = 16384, 32, 8, 3, 9
NC, NS, L = 2, 16, 16
NW = NC * NS
ROWS = B // NW
GROUPS = ROWS // L
TR = T // 8
TB = B // 128
WTB = ROWS // 128
PAR_RE = 32


def _softplus(x):
  t = jnp.exp(-jnp.abs(x))
  z = t / (t + 2.0)
  w = z * z
  poly = 1.0 + w * (1.0 / 3.0 + w * (1.0 / 5.0 + w * (1.0 / 7.0)))
  return jnp.maximum(x, 0.0) + 2.0 * z * poly


def _body(ns_hbm, ch_hbm, rl_hbm, par_hbm, out_hbm,
          ns_v, ch_v, rl_v, par_v, sem_ns, sem_ch, sem_rl):
  cid = lax.axis_index("c")
  sid = lax.axis_index("s")
  wid = sid * NC + cid
  tb0 = wid * WTB

  lane = lax.broadcasted_iota(jnp.int32, (L,), 0)

  cp_ns = pltpu.async_copy(ns_hbm.at[:, :, pl.ds(tb0, WTB)], ns_v, sem_ns)
  cp_ch = pltpu.async_copy(ch_hbm.at[:, pl.ds(tb0, WTB), C - 1], ch_v, sem_ch)
  cp_rl = pltpu.async_copy(rl_hbm.at[:, pl.ds(tb0, WTB), C - 1], rl_v, sem_rl)
  pltpu.sync_copy(par_hbm, par_v)
  cp_ns.wait()
  cp_ch.wait()
  cp_rl.wait()

  wv = par_v[pl.ds(0, L)]
  vv = par_v[pl.ds(L, L)]
  w = [[wv[q * P + p] for p in range(P)] for q in range(P)]
  v = [[vv[q * P + p] for p in range(P)] for q in range(P)]

  def i_step(i, carry):
    itr = i >> 3
    ir = i & 7

    @plsc.parallel_loop(0, GROUPS, unroll=4)
    def _group(g):
      tcl = g >> 3
      l0 = (g & 7) * L
      sl = pl.ds(l0, L)
      tcl_v = jnp.full((L,), 0, jnp.int32) + tcl
      lane_v = l0 + lane
      c = ch_v[i, tcl, sl]
      r = rl_v[i, tcl, sl]
      ctr = lax.shift_right_logical(c, 3)
      cr = jnp.bitwise_and(c, 7)
      prnt = [ns_v[p, itr, tcl, ir, sl] for p in range(P)]
      cs = [plsc.load_gather(
          ns_v, [jnp.full((L,), p, jnp.int32), ctr, tcl_v, cr, lane_v])
            for p in range(P)]
      re = [plsc.load_gather(par_v, [r * P + (PAR_RE + p)]) for p in range(P)]
      a = []
      for p in range(P):
        x = re[p]
        for q in range(P):
          x = x + w[q][p] * prnt[q]
          x = x + v[q][p] * cs[q]
        a.append(_softplus(x) + 1e-6)
      inv = 1.0 / (a[0] + a[1] + a[2])
      for p in range(P):
        ns_v[p, itr, tcl, ir, sl] = a[p] * inv

    return carry

  lax.fori_loop(0, T, i_step, 0)

  pltpu.sync_copy(ns_v.at[:, TR - 1, :, 7, :],
                  out_hbm.at[:, pl.ds(tb0, WTB)])


@jax.jit
def _run(ns5, ch4, rl4, params):
  mesh = plsc.VectorSubcoreMesh(core_axis_name="c", subcore_axis_name="s")
  f = pl.kernel(
      _body,
      out_type=jax.ShapeDtypeStruct((P, TB, 128), jnp.float32),
      mesh=mesh,
      scratch_types=[
          pltpu.VMEM((P, TR, WTB, 8, 128), jnp.float32),
          pltpu.VMEM((T, WTB, 128), jnp.int32),
          pltpu.VMEM((T, WTB, 128), jnp.int32),
          pltpu.VMEM((64,), jnp.float32),
          pltpu.SemaphoreType.DMA,
          pltpu.SemaphoreType.DMA,
          pltpu.SemaphoreType.DMA,
      ],
      compiler_params=pltpu.CompilerParams(
          needs_layout_passes=False, use_tc_tiling_on_sc=False),
  )
  return f(ns5, ch4, rl4, params)


def kernel(node_scores, children, rels, labels, W, V, rel_emb):
  del labels
  ns5 = (node_scores.transpose(2, 1, 0)
         .reshape(P, TR, 8, TB, 128)
         .transpose(0, 1, 3, 2, 4))
  ch4 = (children.astype(jnp.int32).transpose(1, 2, 0)
         .reshape(T, C, TB, 128)
         .transpose(0, 2, 1, 3))
  rl4 = (rels.astype(jnp.int32).transpose(1, 2, 0)
         .reshape(T, C, TB, 128)
         .transpose(0, 2, 1, 3))
  params = (jnp.zeros((64,), jnp.float32)
            .at[0:P * P].set(W.reshape(-1))
            .at[L:L + P * P].set(V.reshape(-1))
            .at[PAR_RE:PAR_RE + R * P].set(rel_emb.reshape(-1)))
  out = _run(ns5, ch4, rl4, params)
  return out.reshape(P, B).transpose(1, 0)
